# Initial kernel scaffold; baseline (speedup 1.0000x reference)
import jax
import jax.numpy as jnp
from jax.experimental import pallas as pl


def kernel(deltas, class_logits, anchors, valid_anchors_indices):
    raise NotImplementedError("write your pallas kernel here")


# speedup vs baseline: 12.3904x; 12.3904x over previous
import functools

import jax
import jax.numpy as jnp
from jax import lax
from jax.experimental import pallas as pl
from jax.experimental.pallas import tpu as pltpu
from jax.experimental.pallas import tpu_sc as plsc

_B = 2
_N = 20000
_NPAD = 20480
_ROWS = 160
_LANES = 128
_OUT = 500
_OUTPAD = 512
_IOU_THR = 0.3
_SCORE_THR = 0.7
_NEG = -1e30

_NW = 32
_PER_W = (_B * _NPAD) // _NW
_CHUNK = 128
_NCHUNK = _PER_W // _CHUNK
_D = 16


def _sc_gather(table, idx):
    mesh = plsc.VectorSubcoreMesh(core_axis_name="c", subcore_axis_name="s")

    @functools.partial(
        pl.kernel,
        out_type=jax.ShapeDtypeStruct((_NW, _NCHUNK, _CHUNK, _D), jnp.float32),
        mesh=mesh,
        scratch_types=[
            pltpu.VMEM((_NCHUNK, _CHUNK), jnp.int32),
            pltpu.VMEM((_NCHUNK, _CHUNK, _D), jnp.float32),
            pltpu.SemaphoreType.DMA,
        ],
        compiler_params=pltpu.CompilerParams(use_tc_tiling_on_sc=False),
    )
    def gather_kernel(table_hbm, idx_hbm, out_hbm, idx_v, rows_v, sem):
        wid = lax.axis_index("s") * 2 + lax.axis_index("c")
        pltpu.sync_copy(idx_hbm.at[wid], idx_v)
        copies = [
            pltpu.async_copy(table_hbm.at[idx_v.at[k]], rows_v.at[k], sem)
            for k in range(_NCHUNK)
        ]
        for c in copies:
            c.wait()
        pltpu.sync_copy(rows_v, out_hbm.at[wid])

    return gather_kernel(table, idx)


def _nms_body(d0, d1, l0, l1, a0, a1, a2, a3, out_ref,
              sw_ref, y1_ref, y2_ref, ar_ref, fg_ref):
    f32 = jnp.float32
    l0v = l0[0]
    l1v = l1[0]
    m = jnp.maximum(l0v, l1v)
    e0 = jnp.exp(l0v - m)
    e1 = jnp.exp(l1v - m)
    fg = e1 / (e0 + e1)

    a0v = a0[0]
    a2v = a2[0]
    h = a2v - a0v
    cy = (a2v + a0v) * f32(0.5)
    dy = d0[0] * f32(0.1)
    dh = d1[0] * f32(0.2)
    cy = cy + dy * h
    h = h * jnp.exp(dh)
    y1 = cy - h * f32(0.5)
    y2 = cy + h * f32(0.5)
    x1p = a1[0]
    x2p = a3[0]

    row_i = lax.broadcasted_iota(jnp.int32, (_ROWS, _LANES), 0)
    lane_i = lax.broadcasted_iota(jnp.int32, (_ROWS, _LANES), 1)
    slot = row_i * _LANES + lane_i

    sw0 = jnp.where((fg >= f32(_SCORE_THR)) & (slot < _N), fg, f32(_NEG))
    areas = jnp.maximum(f32(0.0), y2 - y1) * jnp.maximum(f32(0.0), x2p - x1p)

    sw_ref[...] = sw0
    y1_ref[...] = y1
    y2_ref[...] = y2
    ar_ref[...] = areas
    fg_ref[...] = fg

    lane_row = lax.broadcasted_iota(jnp.int32, (1, _LANES), 1)

    def step(i, carry):
        sw = sw_ref[...]
        mval = jnp.max(sw)
        valid = mval > f32(-1e29)
        eq = sw == mval
        j = jnp.min(jnp.where(eq, slot, jnp.int32(_NPAD)))
        r = j // _LANES
        c = j - r * _LANES
        onehot = (lane_row == c).astype(f32)

        def ext2(ref):
            return jnp.sum(ref[pl.ds(r, 1), :] * onehot)

        def ext3(ref):
            return jnp.sum(ref[0, pl.ds(r, 1), :] * onehot)

        by1 = ext2(y1_ref)
        by2 = ext2(y2_ref)
        bx1 = ext3(a1)
        bx2 = ext3(a3)
        bs = ext2(fg_ref)
        bl0 = ext3(l0)
        bl1 = ext3(l1)

        y1p = y1_ref[...]
        y2p = y2_ref[...]
        yy1 = jnp.maximum(by1, y1p)
        xx1 = jnp.maximum(bx1, x1p)
        yy2 = jnp.minimum(by2, y2p)
        xx2 = jnp.minimum(bx2, x2p)
        inter = jnp.maximum(f32(0.0), yy2 - yy1) * jnp.maximum(f32(0.0), xx2 - xx1)
        barea = jnp.maximum(f32(0.0), by2 - by1) * jnp.maximum(f32(0.0), bx2 - bx1)
        union = barea + ar_ref[...] - inter
        iou = jnp.where(union > f32(0.0), inter / union, f32(0.0))
        suppress = (iou > f32(_IOU_THR)) | (slot == j)
        sw_ref[...] = jnp.where(suppress, f32(_NEG), sw)

        flag = jnp.where(valid, f32(1.0), f32(0.0))

        def oh(k):
            return (lane_row == k).astype(f32)

        row = (oh(0) * by1 + oh(1) * bx1 + oh(2) * by2 + oh(3) * bx2
               + oh(5) * bs + oh(7) * bl0 + oh(8) * bl1
               + oh(4) + oh(6) + oh(9)) * flag
        out_ref[0, pl.ds(i, 1), :] = row
        return carry

    lax.fori_loop(0, _OUT, step, 0)


def _nms_call(d0, d1, l0, l1, a0, a1, a2, a3):
    plane = pl.BlockSpec((1, _ROWS, _LANES), lambda b: (b, 0, 0))
    return pl.pallas_call(
        _nms_body,
        grid=(_B,),
        in_specs=[plane] * 8,
        out_specs=pl.BlockSpec((1, _OUTPAD, _LANES), lambda b: (b, 0, 0)),
        out_shape=jax.ShapeDtypeStruct((_B, _OUTPAD, _LANES), jnp.float32),
        scratch_shapes=[pltpu.VMEM((_ROWS, _LANES), jnp.float32)] * 5,
    )(d0, d1, l0, l1, a0, a1, a2, a3)


def kernel(deltas, class_logits, anchors, valid_anchors_indices):
    table = jnp.concatenate([deltas, class_logits], axis=-1).reshape(_B * _N, 4)
    table = jnp.pad(table, ((0, 0), (0, _D - 4)))
    idx = valid_anchors_indices.astype(jnp.int32)
    idx = idx + (jnp.arange(_B, dtype=jnp.int32) * _N)[:, None]
    idx = jnp.pad(idx, ((0, 0), (0, _NPAD - _N)))
    idx = idx.reshape(_NW, _NCHUNK, _CHUNK)

    gathered = _sc_gather(table, idx)
    g = gathered.reshape(_B, _ROWS, _LANES, _D).transpose(0, 3, 1, 2)
    d0, d1, l0, l1 = g[:, 0], g[:, 1], g[:, 2], g[:, 3]

    ap = jnp.pad(anchors, ((0, 0), (0, _NPAD - _N), (0, 0)))
    a = ap.reshape(_B, _ROWS, _LANES, 4).transpose(0, 3, 1, 2)
    a0, a1, a2, a3 = a[:, 0], a[:, 1], a[:, 2], a[:, 3]

    out = _nms_call(d0, d1, l0, l1, a0, a1, a2, a3)
    boxes = out[:, :_OUT, 0:5]
    scores = out[:, :_OUT, 5:7]
    logits = out[:, :_OUT, 7:10]
    return (boxes, scores, logits)


# speedup vs baseline: 13.7702x; 1.1114x over previous
import functools

import jax
import jax.numpy as jnp
from jax import lax
from jax.experimental import pallas as pl
from jax.experimental.pallas import tpu as pltpu
from jax.experimental.pallas import tpu_sc as plsc

_B = 2
_N = 20000
_NPAD = 20480
_ROWS = 160
_LANES = 128
_OUT = 500
_OUTPAD = 512
_IOU_THR = 0.3
_SCORE_THR = 0.7
_NEG = -1e30

_NW = 32
_PER_W = (_B * _NPAD) // _NW
_CHUNK = 128
_NCHUNK = _PER_W // _CHUNK
_D = 16


def _sc_gather(table, idx):
    mesh = plsc.VectorSubcoreMesh(core_axis_name="c", subcore_axis_name="s")

    @functools.partial(
        pl.kernel,
        out_type=jax.ShapeDtypeStruct((_NW, _NCHUNK, _CHUNK, _D), jnp.float32),
        mesh=mesh,
        scratch_types=[
            pltpu.VMEM((_NCHUNK, _CHUNK), jnp.int32),
            pltpu.VMEM((_NCHUNK, _CHUNK, _D), jnp.float32),
            pltpu.SemaphoreType.DMA,
        ],
        compiler_params=pltpu.CompilerParams(use_tc_tiling_on_sc=False),
    )
    def gather_kernel(table_hbm, idx_hbm, out_hbm, idx_v, rows_v, sem):
        wid = lax.axis_index("s") * 2 + lax.axis_index("c")
        pltpu.sync_copy(idx_hbm.at[wid], idx_v)
        copies = [
            pltpu.async_copy(table_hbm.at[idx_v.at[k]], rows_v.at[k], sem)
            for k in range(_NCHUNK)
        ]
        for c in copies:
            c.wait()
        pltpu.sync_copy(rows_v, out_hbm.at[wid])

    return gather_kernel(table, idx)


def _nms_body(d0, d1, l0, l1, a0, a1, a2, a3, out_ref,
              sw_ref, y1_ref, y2_ref, ar_ref, fg_ref):
    f32 = jnp.float32
    row_i = lax.broadcasted_iota(jnp.int32, (_ROWS, _LANES), 0)
    lane_i = lax.broadcasted_iota(jnp.int32, (_ROWS, _LANES), 1)
    slot = row_i * _LANES + lane_i
    lane_row = lax.broadcasted_iota(jnp.int32, (1, _LANES), 1)

    for b in range(_B):
        l0v = l0[b]
        l1v = l1[b]
        m = jnp.maximum(l0v, l1v)
        e0 = jnp.exp(l0v - m)
        e1 = jnp.exp(l1v - m)
        fg = e1 / (e0 + e1)

        a0v = a0[b]
        a2v = a2[b]
        h = a2v - a0v
        cy = (a2v + a0v) * f32(0.5)
        dy = d0[b] * f32(0.1)
        dh = d1[b] * f32(0.2)
        cy = cy + dy * h
        h = h * jnp.exp(dh)
        y1 = cy - h * f32(0.5)
        y2 = cy + h * f32(0.5)
        x1p = a1[b]
        x2p = a3[b]

        sw0 = jnp.where((fg >= f32(_SCORE_THR)) & (slot < _N), fg, f32(_NEG))
        areas = jnp.maximum(f32(0.0), y2 - y1) * jnp.maximum(f32(0.0), x2p - x1p)

        sw_ref[b] = sw0
        y1_ref[b] = y1
        y2_ref[b] = y2
        ar_ref[b] = areas
        fg_ref[b] = fg

    def step(i, carry):
        for b in range(_B):
            sw = sw_ref[b]
            mval = jnp.max(sw)
            valid = mval > f32(-1e29)
            eq = sw == mval
            j = jnp.min(jnp.where(eq, slot, jnp.int32(_NPAD)))
            r = j // _LANES
            c = j - r * _LANES
            onehot = (lane_row == c).astype(f32)

            def ext(ref):
                return jnp.sum(ref[b, pl.ds(r, 1), :] * onehot)

            by1 = ext(y1_ref)
            by2 = ext(y2_ref)
            bx1 = ext(a1)
            bx2 = ext(a3)
            bs = ext(fg_ref)
            bl0 = ext(l0)
            bl1 = ext(l1)

            y1p = y1_ref[b]
            y2p = y2_ref[b]
            x1p = a1[b]
            x2p = a3[b]
            yy1 = jnp.maximum(by1, y1p)
            xx1 = jnp.maximum(bx1, x1p)
            yy2 = jnp.minimum(by2, y2p)
            xx2 = jnp.minimum(bx2, x2p)
            inter = jnp.maximum(f32(0.0), yy2 - yy1) * jnp.maximum(f32(0.0), xx2 - xx1)
            barea = jnp.maximum(f32(0.0), by2 - by1) * jnp.maximum(f32(0.0), bx2 - bx1)
            union = barea + ar_ref[b] - inter
            iou = jnp.where(union > f32(0.0), inter / union, f32(0.0))
            suppress = (iou > f32(_IOU_THR)) | (slot == j)
            sw_ref[b] = jnp.where(suppress, f32(_NEG), sw)

            flag = jnp.where(valid, f32(1.0), f32(0.0))

            def oh(k):
                return (lane_row == k).astype(f32)

            row = (oh(0) * by1 + oh(1) * bx1 + oh(2) * by2 + oh(3) * bx2
                   + oh(5) * bs + oh(7) * bl0 + oh(8) * bl1
                   + oh(4) + oh(6) + oh(9)) * flag
            out_ref[b, pl.ds(i, 1), :] = row
        return carry

    lax.fori_loop(0, _OUT, step, 0)


def _nms_call(d0, d1, l0, l1, a0, a1, a2, a3):
    return pl.pallas_call(
        _nms_body,
        out_shape=jax.ShapeDtypeStruct((_B, _OUTPAD, _LANES), jnp.float32),
        scratch_shapes=[pltpu.VMEM((_B, _ROWS, _LANES), jnp.float32)] * 5,
    )(d0, d1, l0, l1, a0, a1, a2, a3)


def kernel(deltas, class_logits, anchors, valid_anchors_indices):
    table = jnp.concatenate([deltas, class_logits], axis=-1).reshape(_B * _N, 4)
    table = jnp.pad(table, ((0, 0), (0, _D - 4)))
    idx = valid_anchors_indices.astype(jnp.int32)
    idx = idx + (jnp.arange(_B, dtype=jnp.int32) * _N)[:, None]
    idx = jnp.pad(idx, ((0, 0), (0, _NPAD - _N)))
    idx = idx.reshape(_NW, _NCHUNK, _CHUNK)

    gathered = _sc_gather(table, idx)
    g = gathered.reshape(_B, _ROWS, _LANES, _D).transpose(0, 3, 1, 2)
    d0, d1, l0, l1 = g[:, 0], g[:, 1], g[:, 2], g[:, 3]

    ap = jnp.pad(anchors, ((0, 0), (0, _NPAD - _N), (0, 0)))
    a = ap.reshape(_B, _ROWS, _LANES, 4).transpose(0, 3, 1, 2)
    a0, a1, a2, a3 = a[:, 0], a[:, 1], a[:, 2], a[:, 3]

    out = _nms_call(d0, d1, l0, l1, a0, a1, a2, a3)
    boxes = out[:, :_OUT, 0:5]
    scores = out[:, :_OUT, 5:7]
    logits = out[:, :_OUT, 7:10]
    return (boxes, scores, logits)


# speedup vs baseline: 19.8613x; 1.4423x over previous
import functools

import jax
import jax.numpy as jnp
from jax import lax
from jax.experimental import pallas as pl
from jax.experimental.pallas import tpu as pltpu
from jax.experimental.pallas import tpu_sc as plsc

_B = 2
_N = 20000
_NPAD = 20480
_ROWS = 160
_LANES = 128
_OUT = 500
_OUTPAD = 512
_IOU_THR = 0.3
_SCORE_THR = 0.7
_NEG = -1e30

_NW = 32
_PER_W = (_B * _NPAD) // _NW
_CHUNK = 128
_NCHUNK = _PER_W // _CHUNK
_D = 16


def _sc_gather(table, idx):
    mesh = plsc.VectorSubcoreMesh(core_axis_name="c", subcore_axis_name="s")

    @functools.partial(
        pl.kernel,
        out_type=jax.ShapeDtypeStruct((_NW, _NCHUNK, _CHUNK, _D), jnp.float32),
        mesh=mesh,
        scratch_types=[
            pltpu.VMEM((_NCHUNK, _CHUNK), jnp.int32),
            pltpu.VMEM((_NCHUNK, _CHUNK, _D), jnp.float32),
            pltpu.SemaphoreType.DMA,
        ],
        compiler_params=pltpu.CompilerParams(use_tc_tiling_on_sc=False),
    )
    def gather_kernel(table_hbm, idx_hbm, out_hbm, idx_v, rows_v, sem):
        wid = lax.axis_index("s") * 2 + lax.axis_index("c")
        pltpu.sync_copy(idx_hbm.at[wid], idx_v)
        copies = [
            pltpu.async_copy(table_hbm.at[idx_v.at[k]], rows_v.at[k], sem)
            for k in range(_NCHUNK)
        ]
        for c in copies:
            c.wait()
        pltpu.sync_copy(rows_v, out_hbm.at[wid])

    return gather_kernel(table, idx)


def _nms_body(d0, d1, l0, l1, a0, a1, a2, a3, out_ref,
              sw_ref, y1_ref, y2_ref, ar_ref, fg_ref):
    f32 = jnp.float32
    i32 = jnp.int32
    row_i = lax.broadcasted_iota(i32, (_ROWS, _LANES), 0)
    lane_i = lax.broadcasted_iota(i32, (_ROWS, _LANES), 1)
    slot = row_i * _LANES + lane_i
    lane_row = lax.broadcasted_iota(i32, (1, _LANES), 1)
    ones_mat = jnp.ones((_LANES, _LANES), f32)

    for b in range(_B):
        l0v = l0[b]
        l1v = l1[b]
        m = jnp.maximum(l0v, l1v)
        e0 = jnp.exp(l0v - m)
        e1 = jnp.exp(l1v - m)
        fg = e1 / (e0 + e1)

        a0v = a0[b]
        a2v = a2[b]
        h = a2v - a0v
        cy = (a2v + a0v) * f32(0.5)
        dy = d0[b] * f32(0.1)
        dh = d1[b] * f32(0.2)
        cy = cy + dy * h
        h = h * jnp.exp(dh)
        y1 = cy - h * f32(0.5)
        y2 = cy + h * f32(0.5)
        x1p = a1[b]
        x2p = a3[b]

        sw0 = jnp.where((fg >= f32(_SCORE_THR)) & (slot < _N), fg, f32(_NEG))
        areas = jnp.maximum(f32(0.0), y2 - y1) * jnp.maximum(f32(0.0), x2p - x1p)

        sw_ref[b] = sw0
        y1_ref[b] = y1
        y2_ref[b] = y2
        ar_ref[b] = areas
        fg_ref[b] = fg

    def step(i, carry):
        sws = [sw_ref[b] for b in range(_B)]
        mvals = [jnp.max(sws[b]) for b in range(_B)]
        js = [jnp.min(jnp.where(sws[b] == mvals[b], slot, i32(_NPAD)))
              for b in range(_B)]
        rs = [js[b] // _LANES for b in range(_B)]
        cs = [js[b] - rs[b] * _LANES for b in range(_B)]
        onehots = [(lane_row == cs[b]).astype(f32) for b in range(_B)]
        rows7s = [jnp.concatenate(
            [y1_ref[b, pl.ds(rs[b], 1), :],
             a1[b, pl.ds(rs[b], 1), :],
             y2_ref[b, pl.ds(rs[b], 1), :],
             a3[b, pl.ds(rs[b], 1), :],
             fg_ref[b, pl.ds(rs[b], 1), :],
             l0[b, pl.ds(rs[b], 1), :],
             l1[b, pl.ds(rs[b], 1), :]], axis=0) for b in range(_B)]
        bvalss = [jnp.sum(rows7s[b] * onehots[b], axis=1, keepdims=True)
                  for b in range(_B)]
        for b in range(_B):
            sw = sws[b]
            j = js[b]
            bvals = bvalss[b]
            by1 = bvals[0:1, 0:1]
            bx1 = bvals[1:2, 0:1]
            by2 = bvals[2:3, 0:1]
            bx2 = bvals[3:4, 0:1]
            bs = bvals[4:5, 0:1]
            bl0 = bvals[5:6, 0:1]
            bl1 = bvals[6:7, 0:1]

            y1p = y1_ref[b]
            y2p = y2_ref[b]
            x1p = a1[b]
            x2p = a3[b]
            yy1 = jnp.maximum(by1, y1p)
            xx1 = jnp.maximum(bx1, x1p)
            yy2 = jnp.minimum(by2, y2p)
            xx2 = jnp.minimum(bx2, x2p)
            inter = jnp.maximum(f32(0.0), yy2 - yy1) * jnp.maximum(f32(0.0), xx2 - xx1)
            barea = jnp.maximum(f32(0.0), by2 - by1) * jnp.maximum(f32(0.0), bx2 - bx1)
            union = barea + ar_ref[b] - inter
            iou = jnp.where(union > f32(0.0), inter / union, f32(0.0))
            suppress = (iou > f32(_IOU_THR)) | (slot == j)
            sw_ref[b] = jnp.where(suppress, f32(_NEG), sw)

            flagv = jnp.where(mvals[b] > f32(-1e29), f32(1.0), f32(0.0))

            def oh(k):
                return (lane_row == k).astype(f32)

            row = (oh(0) * by1 + oh(1) * bx1 + oh(2) * by2 + oh(3) * bx2
                   + oh(5) * bs + oh(7) * bl0 + oh(8) * bl1
                   + oh(4) + oh(6) + oh(9)) * flagv
            out_ref[b, pl.ds(i, 1), :] = row
        return carry

    lax.fori_loop(0, _OUT, step, 0)


def _nms_call(d0, d1, l0, l1, a0, a1, a2, a3):
    return pl.pallas_call(
        _nms_body,
        out_shape=jax.ShapeDtypeStruct((_B, _OUTPAD, _LANES), jnp.float32),
        scratch_shapes=[pltpu.VMEM((_B, _ROWS, _LANES), jnp.float32)] * 5,
    )(d0, d1, l0, l1, a0, a1, a2, a3)


def kernel(deltas, class_logits, anchors, valid_anchors_indices):
    table = jnp.concatenate([deltas, class_logits], axis=-1).reshape(_B * _N, 4)
    table = jnp.pad(table, ((0, 0), (0, _D - 4)))
    idx = valid_anchors_indices.astype(jnp.int32)
    idx = idx + (jnp.arange(_B, dtype=jnp.int32) * _N)[:, None]
    idx = jnp.pad(idx, ((0, 0), (0, _NPAD - _N)))
    idx = idx.reshape(_NW, _NCHUNK, _CHUNK)

    gathered = _sc_gather(table, idx)
    g = gathered.reshape(_B, _ROWS, _LANES, _D).transpose(0, 3, 1, 2)
    d0, d1, l0, l1 = g[:, 0], g[:, 1], g[:, 2], g[:, 3]

    ap = jnp.pad(anchors, ((0, 0), (0, _NPAD - _N), (0, 0)))
    a = ap.reshape(_B, _ROWS, _LANES, 4).transpose(0, 3, 1, 2)
    a0, a1, a2, a3 = a[:, 0], a[:, 1], a[:, 2], a[:, 3]

    out = _nms_call(d0, d1, l0, l1, a0, a1, a2, a3)
    boxes = out[:, :_OUT, 0:5]
    scores = out[:, :_OUT, 5:7]
    logits = out[:, :_OUT, 7:10]
    return (boxes, scores, logits)


# speedup vs baseline: 22.1916x; 1.1173x over previous
import functools

import jax
import jax.numpy as jnp
from jax import lax
from jax.experimental import pallas as pl
from jax.experimental.pallas import tpu as pltpu
from jax.experimental.pallas import tpu_sc as plsc

_B = 2
_N = 20000
_NPAD = 20480
_ROWS = 160
_LANES = 128
_OUT = 500
_OUTPAD = 512
_IOU_THR = 0.3
_SCORE_THR = 0.7
_NEG = -1e30

_NW = 32
_PER_W = (_B * _NPAD) // _NW
_CHUNK = 128
_NCHUNK = _PER_W // _CHUNK
_D = 16


def _sc_gather(table, idx):
    mesh = plsc.VectorSubcoreMesh(core_axis_name="c", subcore_axis_name="s")

    @functools.partial(
        pl.kernel,
        out_type=jax.ShapeDtypeStruct((_NW, _NCHUNK, _CHUNK, _D), jnp.float32),
        mesh=mesh,
        scratch_types=[
            pltpu.VMEM((_NCHUNK, _CHUNK), jnp.int32),
            pltpu.VMEM((_NCHUNK, _CHUNK, _D), jnp.float32),
            pltpu.SemaphoreType.DMA,
        ],
        compiler_params=pltpu.CompilerParams(use_tc_tiling_on_sc=False),
    )
    def gather_kernel(table_hbm, idx_hbm, out_hbm, idx_v, rows_v, sem):
        wid = lax.axis_index("s") * 2 + lax.axis_index("c")
        pltpu.sync_copy(idx_hbm.at[wid], idx_v)
        copies = [
            pltpu.async_copy(table_hbm.at[idx_v.at[k]], rows_v.at[k], sem)
            for k in range(_NCHUNK)
        ]
        for c in copies:
            c.wait()
        pltpu.sync_copy(rows_v, out_hbm.at[wid])

    return gather_kernel(table, idx)


def _nms_body(d0, d1, l0, l1, a0, a1, a2, a3, out_ref,
              sw_ref, y1_ref, y2_ref, ar_ref, fg_ref):
    f32 = jnp.float32
    i32 = jnp.int32
    row_i = lax.broadcasted_iota(i32, (_ROWS, _LANES), 0)
    lane_i = lax.broadcasted_iota(i32, (_ROWS, _LANES), 1)
    slot = row_i * _LANES + lane_i
    lane_row = lax.broadcasted_iota(i32, (1, _LANES), 1)
    slotf = slot.astype(f32)
    ones_mat = jnp.ones((_LANES, _LANES), f32)

    for b in range(_B):
        l0v = l0[b]
        l1v = l1[b]
        m = jnp.maximum(l0v, l1v)
        e0 = jnp.exp(l0v - m)
        e1 = jnp.exp(l1v - m)
        fg = e1 / (e0 + e1)

        a0v = a0[b]
        a2v = a2[b]
        h = a2v - a0v
        cy = (a2v + a0v) * f32(0.5)
        dy = d0[b] * f32(0.1)
        dh = d1[b] * f32(0.2)
        cy = cy + dy * h
        h = h * jnp.exp(dh)
        y1 = cy - h * f32(0.5)
        y2 = cy + h * f32(0.5)
        x1p = a1[b]
        x2p = a3[b]

        sw0 = jnp.where((fg >= f32(_SCORE_THR)) & (slot < _N), fg, f32(_NEG))
        areas = jnp.maximum(f32(0.0), y2 - y1) * jnp.maximum(f32(0.0), x2p - x1p)

        sw_ref[b] = sw0
        y1_ref[b] = y1
        y2_ref[b] = y2
        ar_ref[b] = areas
        fg_ref[b] = fg

    def step(i, carry):
        sws = [sw_ref[b] for b in range(_B)]
        mvals = [jnp.max(sws[b]) for b in range(_B)]
        jfs = [jnp.min(jnp.where(sws[b] == mvals[b], slotf, f32(_NPAD)))
               for b in range(_B)]
        js = [jfs[b].astype(i32) for b in range(_B)]
        rs = [js[b] // _LANES for b in range(_B)]
        cs = [js[b] - rs[b] * _LANES for b in range(_B)]
        onehots = [(lane_row == cs[b]).astype(f32) for b in range(_B)]
        rows7s = [jnp.concatenate(
            [y1_ref[b, pl.ds(rs[b], 1), :],
             a1[b, pl.ds(rs[b], 1), :],
             y2_ref[b, pl.ds(rs[b], 1), :],
             a3[b, pl.ds(rs[b], 1), :],
             fg_ref[b, pl.ds(rs[b], 1), :],
             l0[b, pl.ds(rs[b], 1), :],
             l1[b, pl.ds(rs[b], 1), :]], axis=0) for b in range(_B)]
        bvalss = [jnp.sum(rows7s[b] * onehots[b], axis=1, keepdims=True)
                  for b in range(_B)]
        for b in range(_B):
            sw = sws[b]
            j = js[b]
            bvals = bvalss[b]
            by1 = bvals[0:1, 0:1]
            bx1 = bvals[1:2, 0:1]
            by2 = bvals[2:3, 0:1]
            bx2 = bvals[3:4, 0:1]
            bs = bvals[4:5, 0:1]
            bl0 = bvals[5:6, 0:1]
            bl1 = bvals[6:7, 0:1]

            y1p = y1_ref[b]
            y2p = y2_ref[b]
            x1p = a1[b]
            x2p = a3[b]
            yy1 = jnp.maximum(by1, y1p)
            xx1 = jnp.maximum(bx1, x1p)
            yy2 = jnp.minimum(by2, y2p)
            xx2 = jnp.minimum(bx2, x2p)
            inter = jnp.maximum(f32(0.0), yy2 - yy1) * jnp.maximum(f32(0.0), xx2 - xx1)
            barea = jnp.maximum(f32(0.0), by2 - by1) * jnp.maximum(f32(0.0), bx2 - bx1)
            union = barea + ar_ref[b] - inter
            iou = jnp.where(union > f32(0.0), inter / union, f32(0.0))
            suppress = (iou > f32(_IOU_THR)) | (slotf == jfs[b])
            sw_ref[b] = jnp.where(suppress, f32(_NEG), sw)

            flagv = jnp.where(mvals[b] > f32(-1e29), f32(1.0), f32(0.0))

            def oh(k):
                return (lane_row == k).astype(f32)

            row = (oh(0) * by1 + oh(1) * bx1 + oh(2) * by2 + oh(3) * bx2
                   + oh(5) * bs + oh(7) * bl0 + oh(8) * bl1
                   + oh(4) + oh(6) + oh(9)) * flagv
            out_ref[b, pl.ds(i, 1), :] = row
        return carry

    lax.fori_loop(0, _OUT, step, 0)


def _nms_call(d0, d1, l0, l1, a0, a1, a2, a3):
    return pl.pallas_call(
        _nms_body,
        out_shape=jax.ShapeDtypeStruct((_B, _OUTPAD, _LANES), jnp.float32),
        scratch_shapes=[pltpu.VMEM((_B, _ROWS, _LANES), jnp.float32)] * 5,
    )(d0, d1, l0, l1, a0, a1, a2, a3)


def kernel(deltas, class_logits, anchors, valid_anchors_indices):
    table = jnp.concatenate([deltas, class_logits], axis=-1).reshape(_B * _N, 4)
    table = jnp.pad(table, ((0, 0), (0, _D - 4)))
    idx = valid_anchors_indices.astype(jnp.int32)
    idx = idx + (jnp.arange(_B, dtype=jnp.int32) * _N)[:, None]
    idx = jnp.pad(idx, ((0, 0), (0, _NPAD - _N)))
    idx = idx.reshape(_NW, _NCHUNK, _CHUNK)

    gathered = _sc_gather(table, idx)
    g = gathered.reshape(_B, _ROWS, _LANES, _D).transpose(0, 3, 1, 2)
    d0, d1, l0, l1 = g[:, 0], g[:, 1], g[:, 2], g[:, 3]

    ap = jnp.pad(anchors, ((0, 0), (0, _NPAD - _N), (0, 0)))
    a = ap.reshape(_B, _ROWS, _LANES, 4).transpose(0, 3, 1, 2)
    a0, a1, a2, a3 = a[:, 0], a[:, 1], a[:, 2], a[:, 3]

    out = _nms_call(d0, d1, l0, l1, a0, a1, a2, a3)
    boxes = out[:, :_OUT, 0:5]
    scores = out[:, :_OUT, 5:7]
    logits = out[:, :_OUT, 7:10]
    return (boxes, scores, logits)


# speedup vs baseline: 22.4137x; 1.0100x over previous
import functools

import jax
import jax.numpy as jnp
from jax import lax
from jax.experimental import pallas as pl
from jax.experimental.pallas import tpu as pltpu
from jax.experimental.pallas import tpu_sc as plsc

_B = 2
_N = 20000
_NPAD = 20480
_ROWS = 160
_LANES = 128
_OUT = 500
_OUTPAD = 512
_IOU_THR = 0.3
_SCORE_THR = 0.7
_NEG = -1e30

_NW = 32
_PER_W = (_B * _NPAD) // _NW
_CHUNK = 128
_NCHUNK = _PER_W // _CHUNK
_D = 16


def _sc_gather(table, idx):
    mesh = plsc.VectorSubcoreMesh(core_axis_name="c", subcore_axis_name="s")

    @functools.partial(
        pl.kernel,
        out_type=jax.ShapeDtypeStruct((_NW, _NCHUNK, _CHUNK, _D), jnp.float32),
        mesh=mesh,
        scratch_types=[
            pltpu.VMEM((_NCHUNK, _CHUNK), jnp.int32),
            pltpu.VMEM((_NCHUNK, _CHUNK, _D), jnp.float32),
            pltpu.SemaphoreType.DMA,
        ],
        compiler_params=pltpu.CompilerParams(use_tc_tiling_on_sc=False),
    )
    def gather_kernel(table_hbm, idx_hbm, out_hbm, idx_v, rows_v, sem):
        wid = lax.axis_index("s") * 2 + lax.axis_index("c")
        pltpu.sync_copy(idx_hbm.at[wid], idx_v)
        copies = [
            pltpu.async_copy(table_hbm.at[idx_v.at[k]], rows_v.at[k], sem)
            for k in range(_NCHUNK)
        ]
        for c in copies:
            c.wait()
        pltpu.sync_copy(rows_v, out_hbm.at[wid])

    return gather_kernel(table, idx)


def _nms_body(d0, d1, l0, l1, a0, a1, a2, a3, out_ref,
              sw_ref, y1_ref, y2_ref, ar_ref, fg_ref):
    f32 = jnp.float32
    i32 = jnp.int32
    row_i = lax.broadcasted_iota(i32, (_ROWS, _LANES), 0)
    lane_i = lax.broadcasted_iota(i32, (_ROWS, _LANES), 1)
    slot = row_i * _LANES + lane_i
    lane_row = lax.broadcasted_iota(i32, (1, _LANES), 1)
    slotf = slot.astype(f32)
    rowf_i = row_i.astype(f32)
    lanef_row = lane_row.astype(f32)
    ones_mat = jnp.ones((_LANES, _LANES), f32)

    for b in range(_B):
        l0v = l0[b]
        l1v = l1[b]
        m = jnp.maximum(l0v, l1v)
        e0 = jnp.exp(l0v - m)
        e1 = jnp.exp(l1v - m)
        fg = e1 / (e0 + e1)

        a0v = a0[b]
        a2v = a2[b]
        h = a2v - a0v
        cy = (a2v + a0v) * f32(0.5)
        dy = d0[b] * f32(0.1)
        dh = d1[b] * f32(0.2)
        cy = cy + dy * h
        h = h * jnp.exp(dh)
        y1 = cy - h * f32(0.5)
        y2 = cy + h * f32(0.5)
        x1p = a1[b]
        x2p = a3[b]

        sw0 = jnp.where((fg >= f32(_SCORE_THR)) & (slot < _N), fg, f32(_NEG))
        areas = jnp.maximum(f32(0.0), y2 - y1) * jnp.maximum(f32(0.0), x2p - x1p)

        sw_ref[b] = sw0
        y1_ref[b] = y1
        y2_ref[b] = y2
        ar_ref[b] = areas
        fg_ref[b] = fg

    def step(i, carry):
        sws = [sw_ref[b] for b in range(_B)]
        lanemaxs = [jnp.max(sws[b], axis=0, keepdims=True) for b in range(_B)]
        mvals = [jnp.max(lanemaxs[b]) for b in range(_B)]
        rowwinfs = [jnp.min(jnp.where(sws[b] == lanemaxs[b], rowf_i, f32(_ROWS)),
                            axis=0, keepdims=True) for b in range(_B)]
        slotwfs = [rowwinfs[b] * f32(_LANES) + lanef_row for b in range(_B)]
        jfs = [jnp.min(jnp.where(lanemaxs[b] == mvals[b], slotwfs[b], f32(_NPAD)))
               for b in range(_B)]
        js = [jfs[b].astype(i32) for b in range(_B)]
        rs = [js[b] // _LANES for b in range(_B)]
        cs = [js[b] - rs[b] * _LANES for b in range(_B)]
        onehots = [(lane_row == cs[b]).astype(f32) for b in range(_B)]
        rows7s = [jnp.concatenate(
            [y1_ref[b, pl.ds(rs[b], 1), :],
             a1[b, pl.ds(rs[b], 1), :],
             y2_ref[b, pl.ds(rs[b], 1), :],
             a3[b, pl.ds(rs[b], 1), :],
             fg_ref[b, pl.ds(rs[b], 1), :],
             l0[b, pl.ds(rs[b], 1), :],
             l1[b, pl.ds(rs[b], 1), :]], axis=0) for b in range(_B)]
        bvalss = [jnp.sum(rows7s[b] * onehots[b], axis=1, keepdims=True)
                  for b in range(_B)]
        for b in range(_B):
            sw = sws[b]
            j = js[b]
            bvals = bvalss[b]
            by1 = bvals[0:1, 0:1]
            bx1 = bvals[1:2, 0:1]
            by2 = bvals[2:3, 0:1]
            bx2 = bvals[3:4, 0:1]
            bs = bvals[4:5, 0:1]
            bl0 = bvals[5:6, 0:1]
            bl1 = bvals[6:7, 0:1]

            y1p = y1_ref[b]
            y2p = y2_ref[b]
            x1p = a1[b]
            x2p = a3[b]
            yy1 = jnp.maximum(by1, y1p)
            xx1 = jnp.maximum(bx1, x1p)
            yy2 = jnp.minimum(by2, y2p)
            xx2 = jnp.minimum(bx2, x2p)
            inter = jnp.maximum(f32(0.0), yy2 - yy1) * jnp.maximum(f32(0.0), xx2 - xx1)
            barea = jnp.maximum(f32(0.0), by2 - by1) * jnp.maximum(f32(0.0), bx2 - bx1)
            union = barea + ar_ref[b] - inter
            iou = jnp.where(union > f32(0.0), inter / union, f32(0.0))
            suppress = (iou > f32(_IOU_THR)) | (slotf == jfs[b])
            sw_ref[b] = jnp.where(suppress, f32(_NEG), sw)

            flagv = jnp.where(mvals[b] > f32(-1e29), f32(1.0), f32(0.0))

            def oh(k):
                return (lane_row == k).astype(f32)

            row = (oh(0) * by1 + oh(1) * bx1 + oh(2) * by2 + oh(3) * bx2
                   + oh(5) * bs + oh(7) * bl0 + oh(8) * bl1
                   + oh(4) + oh(6) + oh(9)) * flagv
            out_ref[b, pl.ds(i, 1), :] = row
        return carry

    lax.fori_loop(0, _OUT, step, 0)


def _nms_call(d0, d1, l0, l1, a0, a1, a2, a3):
    return pl.pallas_call(
        _nms_body,
        out_shape=jax.ShapeDtypeStruct((_B, _OUTPAD, _LANES), jnp.float32),
        scratch_shapes=[pltpu.VMEM((_B, _ROWS, _LANES), jnp.float32)] * 5,
    )(d0, d1, l0, l1, a0, a1, a2, a3)


def kernel(deltas, class_logits, anchors, valid_anchors_indices):
    table = jnp.concatenate([deltas, class_logits], axis=-1).reshape(_B * _N, 4)
    table = jnp.pad(table, ((0, 0), (0, _D - 4)))
    idx = valid_anchors_indices.astype(jnp.int32)
    idx = idx + (jnp.arange(_B, dtype=jnp.int32) * _N)[:, None]
    idx = jnp.pad(idx, ((0, 0), (0, _NPAD - _N)))
    idx = idx.reshape(_NW, _NCHUNK, _CHUNK)

    gathered = _sc_gather(table, idx)
    g = gathered.reshape(_B, _ROWS, _LANES, _D).transpose(0, 3, 1, 2)
    d0, d1, l0, l1 = g[:, 0], g[:, 1], g[:, 2], g[:, 3]

    ap = jnp.pad(anchors, ((0, 0), (0, _NPAD - _N), (0, 0)))
    a = ap.reshape(_B, _ROWS, _LANES, 4).transpose(0, 3, 1, 2)
    a0, a1, a2, a3 = a[:, 0], a[:, 1], a[:, 2], a[:, 3]

    out = _nms_call(d0, d1, l0, l1, a0, a1, a2, a3)
    boxes = out[:, :_OUT, 0:5]
    scores = out[:, :_OUT, 5:7]
    logits = out[:, :_OUT, 7:10]
    return (boxes, scores, logits)


# speedup vs baseline: 23.7417x; 1.0593x over previous
import functools

import jax
import jax.numpy as jnp
from jax import lax
from jax.experimental import pallas as pl
from jax.experimental.pallas import tpu as pltpu
from jax.experimental.pallas import tpu_sc as plsc

_B = 2
_N = 20000
_NPAD = 20480
_ROWS = 160
_LANES = 128
_OUT = 500
_OUTPAD = 512
_IOU_THR = 0.3
_SCORE_THR = 0.7
_NEG = -1e30

_NW = 32
_PER_W = (_B * _NPAD) // _NW
_CHUNK = 128
_NCHUNK = _PER_W // _CHUNK
_D = 16


def _sc_gather(table, idx):
    mesh = plsc.VectorSubcoreMesh(core_axis_name="c", subcore_axis_name="s")

    @functools.partial(
        pl.kernel,
        out_type=jax.ShapeDtypeStruct((_NW, _NCHUNK, _CHUNK, _D), jnp.float32),
        mesh=mesh,
        scratch_types=[
            pltpu.VMEM((_NCHUNK, _CHUNK), jnp.int32),
            pltpu.VMEM((_NCHUNK, _CHUNK, _D), jnp.float32),
            pltpu.SemaphoreType.DMA,
        ],
        compiler_params=pltpu.CompilerParams(use_tc_tiling_on_sc=False),
    )
    def gather_kernel(table_hbm, idx_hbm, out_hbm, idx_v, rows_v, sem):
        wid = lax.axis_index("s") * 2 + lax.axis_index("c")
        pltpu.sync_copy(idx_hbm.at[wid], idx_v)
        copies = [
            pltpu.async_copy(table_hbm.at[idx_v.at[k]], rows_v.at[k], sem)
            for k in range(_NCHUNK)
        ]
        for c in copies:
            c.wait()
        pltpu.sync_copy(rows_v, out_hbm.at[wid])

    return gather_kernel(table, idx)


def _nms_body(d0, d1, l0, l1, a0, a1, a2, a3, out_ref,
              sw_ref, y1_ref, y2_ref, ar_ref, fg_ref):
    f32 = jnp.float32
    i32 = jnp.int32
    row_i = lax.broadcasted_iota(i32, (_ROWS, _LANES), 0)
    lane_i = lax.broadcasted_iota(i32, (_ROWS, _LANES), 1)
    slot = row_i * _LANES + lane_i
    lane_row = lax.broadcasted_iota(i32, (1, _LANES), 1)
    slotf = slot.astype(f32)
    rowf_i = row_i.astype(f32)
    lanef_row = lane_row.astype(f32)
    ones_mat = jnp.ones((_LANES, _LANES), f32)

    for b in range(_B):
        l0v = l0[b]
        l1v = l1[b]
        m = jnp.maximum(l0v, l1v)
        e0 = jnp.exp(l0v - m)
        e1 = jnp.exp(l1v - m)
        fg = e1 / (e0 + e1)

        a0v = a0[b]
        a2v = a2[b]
        h = a2v - a0v
        cy = (a2v + a0v) * f32(0.5)
        dy = d0[b] * f32(0.1)
        dh = d1[b] * f32(0.2)
        cy = cy + dy * h
        h = h * jnp.exp(dh)
        y1 = cy - h * f32(0.5)
        y2 = cy + h * f32(0.5)
        x1p = a1[b]
        x2p = a3[b]

        sw0 = jnp.where((fg >= f32(_SCORE_THR)) & (slot < _N), fg, f32(_NEG))
        areas = jnp.maximum(f32(0.0), y2 - y1) * jnp.maximum(f32(0.0), x2p - x1p)

        sw_ref[b] = sw0
        y1_ref[b] = y1
        y2_ref[b] = y2
        ar_ref[b] = areas
        fg_ref[b] = fg

    def step(i, carry):
        sws = [sw_ref[b] for b in range(_B)]
        lanemaxs = [jnp.max(sws[b], axis=0, keepdims=True) for b in range(_B)]
        mvals = [jnp.max(lanemaxs[b]) for b in range(_B)]
        rowwinfs = [jnp.min(jnp.where(sws[b] == lanemaxs[b], rowf_i, f32(_ROWS)),
                            axis=0, keepdims=True) for b in range(_B)]
        slotwfs = [rowwinfs[b] * f32(_LANES) + lanef_row for b in range(_B)]
        jfs = [jnp.min(jnp.where(lanemaxs[b] == mvals[b], slotwfs[b], f32(_NPAD)))
               for b in range(_B)]
        js = [jfs[b].astype(i32) for b in range(_B)]
        rs = [js[b] // _LANES for b in range(_B)]
        cs = [js[b] - rs[b] * _LANES for b in range(_B)]
        onehots = [(lane_row == cs[b]).astype(f32) for b in range(_B)]
        rows7s = [jnp.concatenate(
            [y1_ref[b, pl.ds(rs[b], 1), :],
             a1[b, pl.ds(rs[b], 1), :],
             y2_ref[b, pl.ds(rs[b], 1), :],
             a3[b, pl.ds(rs[b], 1), :],
             fg_ref[b, pl.ds(rs[b], 1), :],
             l0[b, pl.ds(rs[b], 1), :],
             l1[b, pl.ds(rs[b], 1), :]], axis=0) for b in range(_B)]
        bvalss = [jnp.sum(rows7s[b] * onehots[b], axis=1, keepdims=True)
                  for b in range(_B)]
        for b in range(_B):
            sw = sws[b]
            j = js[b]
            bvals = bvalss[b]
            by1 = bvals[0:1, 0:1]
            bx1 = bvals[1:2, 0:1]
            by2 = bvals[2:3, 0:1]
            bx2 = bvals[3:4, 0:1]
            bs = bvals[4:5, 0:1]
            bl0 = bvals[5:6, 0:1]
            bl1 = bvals[6:7, 0:1]

            y1p = y1_ref[b]
            y2p = y2_ref[b]
            x1p = a1[b]
            x2p = a3[b]
            yy1 = jnp.maximum(by1, y1p)
            xx1 = jnp.maximum(bx1, x1p)
            yy2 = jnp.minimum(by2, y2p)
            xx2 = jnp.minimum(bx2, x2p)
            inter = jnp.maximum(f32(0.0), yy2 - yy1) * jnp.maximum(f32(0.0), xx2 - xx1)
            barea = jnp.maximum(f32(0.0), by2 - by1) * jnp.maximum(f32(0.0), bx2 - bx1)
            union = barea + ar_ref[b] - inter
            iou = jnp.where(union > f32(0.0), inter / union, f32(0.0))
            suppress = (iou > f32(_IOU_THR)) | (slotf == jfs[b])
            sw_ref[b] = jnp.where(suppress, f32(_NEG), sw)

            flagv = jnp.where(mvals[b] > f32(-1e29), f32(1.0), f32(0.0))

            def oh(k):
                return (lane_row == k).astype(f32)

            row = (oh(0) * by1 + oh(1) * bx1 + oh(2) * by2 + oh(3) * bx2
                   + oh(5) * bs + oh(7) * bl0 + oh(8) * bl1
                   + oh(4) + oh(6) + oh(9)) * flagv
            out_ref[b, pl.ds(i, 1), :] = row
        return carry

    lax.fori_loop(0, _OUT, step, 0, unroll=2)


def _nms_call(d0, d1, l0, l1, a0, a1, a2, a3):
    return pl.pallas_call(
        _nms_body,
        out_shape=jax.ShapeDtypeStruct((_B, _OUTPAD, _LANES), jnp.float32),
        scratch_shapes=[pltpu.VMEM((_B, _ROWS, _LANES), jnp.float32)] * 5,
    )(d0, d1, l0, l1, a0, a1, a2, a3)


def kernel(deltas, class_logits, anchors, valid_anchors_indices):
    table = jnp.concatenate([deltas, class_logits], axis=-1).reshape(_B * _N, 4)
    table = jnp.pad(table, ((0, 0), (0, _D - 4)))
    idx = valid_anchors_indices.astype(jnp.int32)
    idx = idx + (jnp.arange(_B, dtype=jnp.int32) * _N)[:, None]
    idx = jnp.pad(idx, ((0, 0), (0, _NPAD - _N)))
    idx = idx.reshape(_NW, _NCHUNK, _CHUNK)

    gathered = _sc_gather(table, idx)
    g = gathered.reshape(_B, _ROWS, _LANES, _D).transpose(0, 3, 1, 2)
    d0, d1, l0, l1 = g[:, 0], g[:, 1], g[:, 2], g[:, 3]

    ap = jnp.pad(anchors, ((0, 0), (0, _NPAD - _N), (0, 0)))
    a = ap.reshape(_B, _ROWS, _LANES, 4).transpose(0, 3, 1, 2)
    a0, a1, a2, a3 = a[:, 0], a[:, 1], a[:, 2], a[:, 3]

    out = _nms_call(d0, d1, l0, l1, a0, a1, a2, a3)
    boxes = out[:, :_OUT, 0:5]
    scores = out[:, :_OUT, 5:7]
    logits = out[:, :_OUT, 7:10]
    return (boxes, scores, logits)


# speedup vs baseline: 24.2962x; 1.0234x over previous
import functools

import jax
import jax.numpy as jnp
from jax import lax
from jax.experimental import pallas as pl
from jax.experimental.pallas import tpu as pltpu
from jax.experimental.pallas import tpu_sc as plsc

_B = 2
_N = 20000
_NPAD = 20480
_ROWS = 160
_LANES = 128
_OUT = 500
_OUTPAD = 512
_IOU_THR = 0.3
_SCORE_THR = 0.7
_NEG = -1e30

_NW = 32
_PER_W = (_B * _NPAD) // _NW
_CHUNK = 128
_NCHUNK = _PER_W // _CHUNK
_D = 16


def _sc_gather(table, idx):
    mesh = plsc.VectorSubcoreMesh(core_axis_name="c", subcore_axis_name="s")

    @functools.partial(
        pl.kernel,
        out_type=jax.ShapeDtypeStruct((_NW, _NCHUNK, _CHUNK, _D), jnp.float32),
        mesh=mesh,
        scratch_types=[
            pltpu.VMEM((_NCHUNK, _CHUNK), jnp.int32),
            pltpu.VMEM((_NCHUNK, _CHUNK, _D), jnp.float32),
            pltpu.SemaphoreType.DMA,
        ],
        compiler_params=pltpu.CompilerParams(use_tc_tiling_on_sc=False),
    )
    def gather_kernel(table_hbm, idx_hbm, out_hbm, idx_v, rows_v, sem):
        wid = lax.axis_index("s") * 2 + lax.axis_index("c")
        pltpu.sync_copy(idx_hbm.at[wid], idx_v)
        copies = [
            pltpu.async_copy(table_hbm.at[idx_v.at[k]], rows_v.at[k], sem)
            for k in range(_NCHUNK)
        ]
        for c in copies:
            c.wait()
        pltpu.sync_copy(rows_v, out_hbm.at[wid])

    return gather_kernel(table, idx)


def _nms_body(d0, d1, l0, l1, a0, a1, a2, a3, out_ref,
              sw_ref, y1_ref, y2_ref, ar_ref, fg_ref):
    f32 = jnp.float32
    i32 = jnp.int32
    row_i = lax.broadcasted_iota(i32, (_ROWS, _LANES), 0)
    lane_i = lax.broadcasted_iota(i32, (_ROWS, _LANES), 1)
    slot = row_i * _LANES + lane_i
    lane_row = lax.broadcasted_iota(i32, (1, _LANES), 1)
    slotf = slot.astype(f32)
    rowf_i = row_i.astype(f32)
    lanef_row = lane_row.astype(f32)
    ones_mat = jnp.ones((_LANES, _LANES), f32)

    for b in range(_B):
        l0v = l0[b]
        l1v = l1[b]
        m = jnp.maximum(l0v, l1v)
        e0 = jnp.exp(l0v - m)
        e1 = jnp.exp(l1v - m)
        fg = e1 / (e0 + e1)

        a0v = a0[b]
        a2v = a2[b]
        h = a2v - a0v
        cy = (a2v + a0v) * f32(0.5)
        dy = d0[b] * f32(0.1)
        dh = d1[b] * f32(0.2)
        cy = cy + dy * h
        h = h * jnp.exp(dh)
        y1 = cy - h * f32(0.5)
        y2 = cy + h * f32(0.5)
        x1p = a1[b]
        x2p = a3[b]

        sw0 = jnp.where((fg >= f32(_SCORE_THR)) & (slot < _N), fg, f32(_NEG))
        areas = jnp.maximum(f32(0.0), y2 - y1) * jnp.maximum(f32(0.0), x2p - x1p)

        sw_ref[b] = sw0
        y1_ref[b] = y1
        y2_ref[b] = y2
        ar_ref[b] = areas
        fg_ref[b] = fg

    def step(i, carry):
        sws = [sw_ref[b] for b in range(_B)]
        lanemaxs = [jnp.max(sws[b], axis=0, keepdims=True) for b in range(_B)]
        mvals = [jnp.max(lanemaxs[b]) for b in range(_B)]
        rowwinfs = [jnp.min(jnp.where(sws[b] == lanemaxs[b], rowf_i, f32(_ROWS)),
                            axis=0, keepdims=True) for b in range(_B)]
        slotwfs = [rowwinfs[b] * f32(_LANES) + lanef_row for b in range(_B)]
        jfs = [jnp.min(jnp.where(lanemaxs[b] == mvals[b], slotwfs[b], f32(_NPAD)))
               for b in range(_B)]
        js = [jfs[b].astype(i32) for b in range(_B)]
        rs = [js[b] // _LANES for b in range(_B)]
        cs = [js[b] - rs[b] * _LANES for b in range(_B)]
        onehots = [(lane_row == cs[b]).astype(f32) for b in range(_B)]
        rows7s = [jnp.concatenate(
            [y1_ref[b, pl.ds(rs[b], 1), :],
             a1[b, pl.ds(rs[b], 1), :],
             y2_ref[b, pl.ds(rs[b], 1), :],
             a3[b, pl.ds(rs[b], 1), :],
             fg_ref[b, pl.ds(rs[b], 1), :],
             l0[b, pl.ds(rs[b], 1), :],
             l1[b, pl.ds(rs[b], 1), :]], axis=0) for b in range(_B)]
        bvalss = [jnp.sum(rows7s[b] * onehots[b], axis=1, keepdims=True)
                  for b in range(_B)]
        for b in range(_B):
            sw = sws[b]
            j = js[b]
            bvals = bvalss[b]
            by1 = bvals[0:1, 0:1]
            bx1 = bvals[1:2, 0:1]
            by2 = bvals[2:3, 0:1]
            bx2 = bvals[3:4, 0:1]
            bs = bvals[4:5, 0:1]
            bl0 = bvals[5:6, 0:1]
            bl1 = bvals[6:7, 0:1]

            y1p = y1_ref[b]
            y2p = y2_ref[b]
            x1p = a1[b]
            x2p = a3[b]
            yy1 = jnp.maximum(by1, y1p)
            xx1 = jnp.maximum(bx1, x1p)
            yy2 = jnp.minimum(by2, y2p)
            xx2 = jnp.minimum(bx2, x2p)
            inter = jnp.maximum(f32(0.0), yy2 - yy1) * jnp.maximum(f32(0.0), xx2 - xx1)
            barea = jnp.maximum(f32(0.0), by2 - by1) * jnp.maximum(f32(0.0), bx2 - bx1)
            union = barea + ar_ref[b] - inter
            iou = jnp.where(union > f32(0.0), inter / union, f32(0.0))
            suppress = (iou > f32(_IOU_THR)) | (slotf == jfs[b])
            sw_ref[b] = jnp.where(suppress, f32(_NEG), sw)

            flagv = jnp.where(mvals[b] > f32(-1e29), f32(1.0), f32(0.0))

            def oh(k):
                return (lane_row == k).astype(f32)

            row = (oh(0) * by1 + oh(1) * bx1 + oh(2) * by2 + oh(3) * bx2
                   + oh(5) * bs + oh(7) * bl0 + oh(8) * bl1
                   + oh(4) + oh(6) + oh(9)) * flagv
            out_ref[b, pl.ds(i, 1), :] = row
        return carry

    lax.fori_loop(0, _OUT, step, 0, unroll=4)


def _nms_call(d0, d1, l0, l1, a0, a1, a2, a3):
    return pl.pallas_call(
        _nms_body,
        out_shape=jax.ShapeDtypeStruct((_B, _OUTPAD, _LANES), jnp.float32),
        scratch_shapes=[pltpu.VMEM((_B, _ROWS, _LANES), jnp.float32)] * 5,
    )(d0, d1, l0, l1, a0, a1, a2, a3)


def kernel(deltas, class_logits, anchors, valid_anchors_indices):
    table = jnp.concatenate([deltas, class_logits], axis=-1).reshape(_B * _N, 4)
    table = jnp.pad(table, ((0, 0), (0, _D - 4)))
    idx = valid_anchors_indices.astype(jnp.int32)
    idx = idx + (jnp.arange(_B, dtype=jnp.int32) * _N)[:, None]
    idx = jnp.pad(idx, ((0, 0), (0, _NPAD - _N)))
    idx = idx.reshape(_NW, _NCHUNK, _CHUNK)

    gathered = _sc_gather(table, idx)
    g = gathered.reshape(_B, _ROWS, _LANES, _D).transpose(0, 3, 1, 2)
    d0, d1, l0, l1 = g[:, 0], g[:, 1], g[:, 2], g[:, 3]

    ap = jnp.pad(anchors, ((0, 0), (0, _NPAD - _N), (0, 0)))
    a = ap.reshape(_B, _ROWS, _LANES, 4).transpose(0, 3, 1, 2)
    a0, a1, a2, a3 = a[:, 0], a[:, 1], a[:, 2], a[:, 3]

    out = _nms_call(d0, d1, l0, l1, a0, a1, a2, a3)
    boxes = out[:, :_OUT, 0:5]
    scores = out[:, :_OUT, 5:7]
    logits = out[:, :_OUT, 7:10]
    return (boxes, scores, logits)


# speedup vs baseline: 28.2907x; 1.1644x over previous
import functools

import jax
import jax.numpy as jnp
from jax import lax
from jax.experimental import pallas as pl
from jax.experimental.pallas import tpu as pltpu
from jax.experimental.pallas import tpu_sc as plsc

_B = 2
_N = 20000
_NPAD = 20480
_ROWS = 160
_LANES = 128
_OUT = 500
_OUTPAD = 512
_IOU_THR = 0.3
_SCORE_THR = 0.7
_NEG = -1e30

_NW = 32
_PER_W = (_B * _NPAD) // _NW
_CHUNK = 128
_NCHUNK = _PER_W // _CHUNK


def _sc_gather(tables, idx):
    mesh = plsc.VectorSubcoreMesh(core_axis_name="c", subcore_axis_name="s")

    @functools.partial(
        pl.kernel,
        out_type=jax.ShapeDtypeStruct((4, _NW, _NCHUNK, _CHUNK), jnp.float32),
        mesh=mesh,
        scratch_types=[
            pltpu.VMEM((_NCHUNK, _CHUNK), jnp.int32),
            pltpu.VMEM((4, _NCHUNK, _CHUNK), jnp.float32),
            pltpu.SemaphoreType.DMA,
        ],
        compiler_params=pltpu.CompilerParams(use_tc_tiling_on_sc=False),
    )
    def gather_kernel(tables_hbm, idx_hbm, out_hbm, idx_v, rows_v, sem):
        wid = lax.axis_index("s") * 2 + lax.axis_index("c")
        pltpu.sync_copy(idx_hbm.at[wid], idx_v)
        copies = []
        for d in range(4):
            for k in range(_NCHUNK):
                copies.append(pltpu.async_copy(
                    tables_hbm.at[d].at[idx_v.at[k]],
                    rows_v.at[d].at[k], sem))
        for c in copies:
            c.wait()
        pltpu.sync_copy(rows_v, out_hbm.at[:, wid])

    return gather_kernel(tables, idx)


def _nms_body(d0, d1, l0, l1, a0, a1, a2, a3, out_ref,
              sw_ref, y1_ref, y2_ref, ar_ref, fg_ref):
    f32 = jnp.float32
    i32 = jnp.int32
    row_i = lax.broadcasted_iota(i32, (_ROWS, _LANES), 0)
    lane_i = lax.broadcasted_iota(i32, (_ROWS, _LANES), 1)
    slot = row_i * _LANES + lane_i
    lane_row = lax.broadcasted_iota(i32, (1, _LANES), 1)
    slotf = slot.astype(f32)
    rowf_i = row_i.astype(f32)
    lanef_row = lane_row.astype(f32)
    ones_mat = jnp.ones((_LANES, _LANES), f32)

    for b in range(_B):
        l0v = l0[b]
        l1v = l1[b]
        m = jnp.maximum(l0v, l1v)
        e0 = jnp.exp(l0v - m)
        e1 = jnp.exp(l1v - m)
        fg = e1 / (e0 + e1)

        a0v = a0[b]
        a2v = a2[b]
        h = a2v - a0v
        cy = (a2v + a0v) * f32(0.5)
        dy = d0[b] * f32(0.1)
        dh = d1[b] * f32(0.2)
        cy = cy + dy * h
        h = h * jnp.exp(dh)
        y1 = cy - h * f32(0.5)
        y2 = cy + h * f32(0.5)
        x1p = a1[b]
        x2p = a3[b]

        sw0 = jnp.where((fg >= f32(_SCORE_THR)) & (slot < _N), fg, f32(_NEG))
        areas = jnp.maximum(f32(0.0), y2 - y1) * jnp.maximum(f32(0.0), x2p - x1p)

        sw_ref[b] = sw0
        y1_ref[b] = y1
        y2_ref[b] = y2
        ar_ref[b] = areas
        fg_ref[b] = fg

    def step(i, carry):
        sws = [sw_ref[b] for b in range(_B)]
        lanemaxs = [jnp.max(sws[b], axis=0, keepdims=True) for b in range(_B)]
        mvals = [jnp.max(lanemaxs[b]) for b in range(_B)]
        rowwinfs = [jnp.min(jnp.where(sws[b] == lanemaxs[b], rowf_i, f32(_ROWS)),
                            axis=0, keepdims=True) for b in range(_B)]
        slotwfs = [rowwinfs[b] * f32(_LANES) + lanef_row for b in range(_B)]
        jfs = [jnp.min(jnp.where(lanemaxs[b] == mvals[b], slotwfs[b], f32(_NPAD)))
               for b in range(_B)]
        js = [jfs[b].astype(i32) for b in range(_B)]
        rs = [js[b] // _LANES for b in range(_B)]
        cs = [js[b] - rs[b] * _LANES for b in range(_B)]
        onehots = [(lane_row == cs[b]).astype(f32) for b in range(_B)]
        rows7s = [jnp.concatenate(
            [y1_ref[b, pl.ds(rs[b], 1), :],
             a1[b, pl.ds(rs[b], 1), :],
             y2_ref[b, pl.ds(rs[b], 1), :],
             a3[b, pl.ds(rs[b], 1), :],
             fg_ref[b, pl.ds(rs[b], 1), :],
             l0[b, pl.ds(rs[b], 1), :],
             l1[b, pl.ds(rs[b], 1), :]], axis=0) for b in range(_B)]
        bvalss = [jnp.sum(rows7s[b] * onehots[b], axis=1, keepdims=True)
                  for b in range(_B)]
        for b in range(_B):
            sw = sws[b]
            j = js[b]
            bvals = bvalss[b]
            by1 = bvals[0:1, 0:1]
            bx1 = bvals[1:2, 0:1]
            by2 = bvals[2:3, 0:1]
            bx2 = bvals[3:4, 0:1]
            bs = bvals[4:5, 0:1]
            bl0 = bvals[5:6, 0:1]
            bl1 = bvals[6:7, 0:1]

            y1p = y1_ref[b]
            y2p = y2_ref[b]
            x1p = a1[b]
            x2p = a3[b]
            yy1 = jnp.maximum(by1, y1p)
            xx1 = jnp.maximum(bx1, x1p)
            yy2 = jnp.minimum(by2, y2p)
            xx2 = jnp.minimum(bx2, x2p)
            inter = jnp.maximum(f32(0.0), yy2 - yy1) * jnp.maximum(f32(0.0), xx2 - xx1)
            barea = jnp.maximum(f32(0.0), by2 - by1) * jnp.maximum(f32(0.0), bx2 - bx1)
            union = barea + ar_ref[b] - inter
            iou = jnp.where(union > f32(0.0), inter / union, f32(0.0))
            suppress = (iou > f32(_IOU_THR)) | (slotf == jfs[b])
            sw_ref[b] = jnp.where(suppress, f32(_NEG), sw)

            flagv = jnp.where(mvals[b] > f32(-1e29), f32(1.0), f32(0.0))

            def oh(k):
                return (lane_row == k).astype(f32)

            row = (oh(0) * by1 + oh(1) * bx1 + oh(2) * by2 + oh(3) * bx2
                   + oh(5) * bs + oh(7) * bl0 + oh(8) * bl1
                   + oh(4) + oh(6) + oh(9)) * flagv
            out_ref[b, pl.ds(i, 1), :] = row
        return carry

    lax.fori_loop(0, _OUT, step, 0, unroll=4)


def _nms_call(d0, d1, l0, l1, a0, a1, a2, a3):
    return pl.pallas_call(
        _nms_body,
        out_shape=jax.ShapeDtypeStruct((_B, _OUTPAD, _LANES), jnp.float32),
        scratch_shapes=[pltpu.VMEM((_B, _ROWS, _LANES), jnp.float32)] * 5,
    )(d0, d1, l0, l1, a0, a1, a2, a3)


def kernel(deltas, class_logits, anchors, valid_anchors_indices):
    tables = jnp.stack(
        [deltas[..., 0], deltas[..., 1],
         class_logits[..., 0], class_logits[..., 1]]).reshape(4, _B * _N)
    idx = valid_anchors_indices.astype(jnp.int32)
    idx = idx + (jnp.arange(_B, dtype=jnp.int32) * _N)[:, None]
    idx = jnp.pad(idx, ((0, 0), (0, _NPAD - _N)))
    idx = idx.reshape(_NW, _NCHUNK, _CHUNK)

    gathered = _sc_gather(tables, idx)
    g = gathered.reshape(4, _B, _ROWS, _LANES)
    d0, d1, l0, l1 = g[0], g[1], g[2], g[3]

    ap = jnp.pad(anchors, ((0, 0), (0, _NPAD - _N), (0, 0)))
    a = ap.reshape(_B, _ROWS, _LANES, 4).transpose(0, 3, 1, 2)
    a0, a1, a2, a3 = a[:, 0], a[:, 1], a[:, 2], a[:, 3]

    out = _nms_call(d0, d1, l0, l1, a0, a1, a2, a3)
    boxes = out[:, :_OUT, 0:5]
    scores = out[:, :_OUT, 5:7]
    logits = out[:, :_OUT, 7:10]
    return (boxes, scores, logits)
